# baseline (speedup 1.0000x reference)
import functools

import jax
import jax.numpy as jnp
from jax import lax
from jax.experimental import pallas as pl
from jax.experimental.pallas import tpu as pltpu
from jax.experimental.pallas import tpu_sc as plsc

_MAX_NORM = 1.0 - 1e-05
_MAXSQ = _MAX_NORM * _MAX_NORM
_M = 16
_L = 16
_NW = 32
_ROWS = 4096 * 50
_RPW = _ROWS // _NW
_CH = 128
_NCH = _RPW // _CH
_GRP = 10
_NGRP = _NCH // _GRP

_mesh = plsc.VectorSubcoreMesh(core_axis_name="c", subcore_axis_name="s")


@functools.partial(
    pl.kernel,
    mesh=_mesh,
    out_type=jax.ShapeDtypeStruct((_ROWS, _M), jnp.float32),
    scratch_types=[
        pltpu.VMEM((_NCH, _CH), jnp.int32),
        pltpu.VMEM((_RPW, _M), jnp.float32),
        pltpu.SemaphoreType.DMA,
    ],
    compiler_params=pltpu.CompilerParams(
        needs_layout_passes=False, use_tc_tiling_on_sc=False
    ),
)
def _emb(x_hbm, w_hbm, out_hbm, idx_v, rows_v, sem):
    wid = lax.axis_index("s") * 2 + lax.axis_index("c")
    pltpu.sync_copy(x_hbm.at[wid], idx_v)

    def dma_group(g, carry):
        c0 = g * _GRP
        cps = []
        for j in range(_GRP):
            c = c0 + j
            cps.append(
                pltpu.async_copy(
                    w_hbm.at[idx_v.at[c]],
                    rows_v.at[pl.ds(c * _CH, _CH)],
                    sem,
                )
            )
        for cp in cps:
            cp.wait()
        return carry

    lax.fori_loop(0, _NGRP, dma_group, 0)

    def renorm(gi, carry):
        ri = gi * _L + lax.iota(jnp.int32, _L)
        acc = jnp.zeros((_L,), jnp.float32)
        for j in range(_M):
            cj = jnp.full((_L,), j, jnp.int32)
            col = plsc.load_gather(rows_v, [ri, cj])
            acc = acc + col * col
        bits = plsc.bitcast(acc, jnp.int32)
        y = plsc.bitcast(jnp.int32(0x5F3759DF) - (bits >> 1), jnp.float32)
        for _it in range(3):
            y = y * (1.5 - 0.5 * acc * y * y)
        scale = jnp.where(acc > _MAXSQ, _MAX_NORM * y, 1.0)
        for j in range(_M):
            cj = jnp.full((_L,), j, jnp.int32)
            col = plsc.load_gather(rows_v, [ri, cj])
            plsc.store_scatter(rows_v, [ri, cj], col * scale)
        return carry

    lax.fori_loop(0, _RPW // _L, renorm, 0)

    pltpu.sync_copy(rows_v, out_hbm.at[pl.ds(wid * _RPW, _RPW)])


def kernel(x, weight):
    xf = x.reshape(_NW, _NCH, _CH)
    out = _emb(xf, weight)
    return out.reshape(x.shape[0], x.shape[1], _M)


# speedup vs baseline: 1.4943x; 1.4943x over previous
import functools

import jax
import jax.numpy as jnp
from jax import lax
from jax.experimental import pallas as pl
from jax.experimental.pallas import tpu as pltpu
from jax.experimental.pallas import tpu_sc as plsc

_MAX_NORM = 1.0 - 1e-05
_MAXSQ = _MAX_NORM * _MAX_NORM
_M = 16
_L = 16
_B = 4096
_S = 50
_NW = 32
_TB = 512
_NTASK = _S * (_B // _TB)
_CH = 128
_NCH = _TB // _CH
_KMAX = (_NTASK + _NW - 1) // _NW

_mesh = plsc.VectorSubcoreMesh(core_axis_name="c", subcore_axis_name="s")

_N = 1000000
_TCOLS = _N // 128
_TAIL = _N - _TCOLS * 128


@functools.partial(
    pl.kernel,
    mesh=_mesh,
    out_type=jax.ShapeDtypeStruct((_N * _M // 128, 128), jnp.float32),
    scratch_types=[
        pltpu.VMEM((2 * _M, 128), jnp.float32),
        pltpu.VMEM((2 * _M, 128), jnp.float32),
        pltpu.SemaphoreType.DMA((2,)),
        pltpu.SemaphoreType.DMA((2,)),
    ],
    compiler_params=pltpu.CompilerParams(
        needs_layout_passes=False, use_tc_tiling_on_sc=True
    ),
)
def _transpose(wT_hbm, tail_hbm, out_hbm, src_v, dst_v, sem_i, sem_o):
    wid = lax.axis_index("s") * 2 + lax.axis_index("c")
    nfull = (_TCOLS - wid + _NW - 1) // _NW

    def stage(i, p):
        c = wid + i * _NW
        pltpu.async_copy(
            wT_hbm.at[:, pl.ds(c * 128, 128)],
            src_v.at[pl.ds(p * _M, _M), :],
            sem_i.at[p],
        )

    @pl.when(nfull > 0)
    def _():
        stage(0, 0)

    def body(i, carry):
        c = wid + i * _NW
        p = lax.rem(i, 2)
        p2 = lax.rem(i + 1, 2)

        @pl.when(i + 1 < nfull)
        def _():
            stage(i + 1, p2)

        pltpu.make_async_copy(
            wT_hbm.at[:, pl.ds(0, 128)],
            src_v.at[pl.ds(0, _M), :],
            sem_i.at[p],
        ).wait()

        @pl.when(i >= 2)
        def _():
            pltpu.make_async_copy(
                dst_v.at[pl.ds(0, _M), :],
                out_hbm.at[pl.ds(0, _M), :],
                sem_o.at[p],
            ).wait()

        for a in range(_M):
            for m in range(8):
                ri = lax.iota(jnp.int32, _L) + p * _M
                cj = jnp.full((_L,), 8 * a + m, jnp.int32)
                dst_v[p * _M + a, pl.ds(16 * m, 16)] = plsc.load_gather(
                    src_v, [ri, cj]
                )
        pltpu.async_copy(
            dst_v.at[pl.ds(p * _M, _M), :],
            out_hbm.at[pl.ds(c * _M, _M), :],
            sem_o.at[p],
        )
        return carry

    lax.fori_loop(0, nfull, body, 0)

    def drain(p):
        pltpu.make_async_copy(
            dst_v.at[pl.ds(0, _M), :],
            out_hbm.at[pl.ds(0, _M), :],
            sem_o.at[p],
        ).wait()

    @pl.when(nfull >= 2)
    def _():
        drain(lax.rem(nfull, 2))

    @pl.when(nfull >= 1)
    def _():
        drain(lax.rem(nfull + 1, 2))

    @pl.when(wid == _NW - 1)
    def _():
        pltpu.sync_copy(tail_hbm, src_v.at[pl.ds(0, 8), :])
        pltpu.sync_copy(src_v.at[pl.ds(0, 8), :],
                        out_hbm.at[pl.ds(_TCOLS * _M, 8), :])


@functools.partial(
    pl.kernel,
    mesh=_mesh,
    out_type=jax.ShapeDtypeStruct((_S, _M, _B), jnp.float32),
    scratch_types=[
        pltpu.VMEM((2 * _TB,), jnp.int32),
        pltpu.VMEM((2 * _TB, _M), jnp.float32),
        pltpu.VMEM((2 * _M, _TB), jnp.float32),
        pltpu.SemaphoreType.DMA((2,)),
    ],
    compiler_params=pltpu.CompilerParams(
        needs_layout_passes=False, use_tc_tiling_on_sc=False
    ),
)
def _emb(xT_hbm, w_hbm, out_hbm, idx_v, rows_v, outT_v, sem):
    wid = lax.axis_index("s") * 2 + lax.axis_index("c")

    def stage(t, p):
        l = t // (_B // _TB)
        b0 = (t % (_B // _TB)) * _TB
        pltpu.sync_copy(xT_hbm.at[l, pl.ds(b0, _TB)],
                        idx_v.at[pl.ds(p * _TB, _TB)])
        for c in range(_NCH):
            pltpu.async_copy(
                w_hbm.at[idx_v.at[pl.ds(p * _TB + c * _CH, _CH)]],
                rows_v.at[pl.ds(p * _TB + c * _CH, _CH)],
                sem.at[p],
            )

    stage(wid, 0)

    def body(k, carry):
        t = wid + _NW * k
        p = lax.rem(k, 2)
        t2 = t + _NW
        p2 = lax.rem(k + 1, 2)

        @pl.when(t2 < _NTASK)
        def _():
            stage(t2, p2)

        @pl.when(t < _NTASK)
        def _():
            for c in range(_NCH):
                pltpu.make_async_copy(
                    w_hbm.at[idx_v.at[pl.ds(0, _CH)]],
                    rows_v.at[pl.ds(0, _CH)],
                    sem.at[p],
                ).wait()
            for g in range(_TB // _L):
                rbase = p * _TB + g * _L
                ri = rbase + lax.iota(jnp.int32, _L)
                cols = []
                acc = jnp.zeros((_L,), jnp.float32)
                for j in range(_M):
                    cj = jnp.full((_L,), j, jnp.int32)
                    col = plsc.load_gather(rows_v, [ri, cj])
                    cols.append(col)
                    acc = acc + col * col
                bits = plsc.bitcast(acc, jnp.int32)
                y = plsc.bitcast(jnp.int32(0x5F3759DF) - (bits >> 1),
                                 jnp.float32)
                for _it in range(3):
                    y = y * (1.5 - 0.5 * acc * y * y)
                scale = jnp.where(acc > _MAXSQ, _MAX_NORM * y, 1.0)
                for j in range(_M):
                    outT_v[p * _M + j, pl.ds(g * _L, _L)] = cols[j] * scale
            l = t // (_B // _TB)
            b0 = (t % (_B // _TB)) * _TB
            pltpu.sync_copy(outT_v.at[pl.ds(p * _M, _M), :],
                            out_hbm.at[l, :, pl.ds(b0, _TB)])

        return carry

    lax.fori_loop(0, _KMAX, body, 0)


def kernel(x, weight):
    tail = weight[_TCOLS * 128:, :].reshape(8, 128)
    w_lin = _transpose(weight.T, tail)
    out = _emb(x.T, w_lin.reshape(_N, _M))
    return jnp.transpose(out, (2, 0, 1))
